# Initial kernel scaffold; baseline (speedup 1.0000x reference)
import jax
import jax.numpy as jnp
from jax.experimental import pallas as pl


def kernel(x, edge_index, edge_weight, ln_g4, ln_b4, W4, b4, ln_g5, ln_b5, W5, b5, ln_g6, ln_b6, W6, b6, ln_g7, ln_b7, W7, b7, W_post, b_post):
    raise NotImplementedError("write your pallas kernel here")


# speedup vs baseline: 3.5719x; 3.5719x over previous
import functools

import jax
import jax.numpy as jnp
from jax import lax
from jax.experimental import pallas as pl
from jax.experimental.pallas import tpu as pltpu
from jax.experimental.pallas import tpu_sc as plsc

N = 10000
E = 320000
D = 128

NC = 2
NS = 16
NW = NC * NS
C = 128
NCHUNK = E // C
CHUNK_ITERS = -(-NCHUNK // NW)
ROWS_PER_SUB = 624
TAIL_OFF = NS * ROWS_PER_SUB
TAIL_ROWS = N - TAIL_OFF

BLK = 1000
GRID = N // BLK


def _sc_agg_body(xn_hbm, src_hbm, dst_hbm, ewb_hbm, out_hbm,
                 src_v, dst_v, w16_v, rows_v, accum, sem):
    cid = lax.axis_index("c")
    sid = lax.axis_index("s")
    wid = sid * NC + cid

    zero16 = jnp.zeros((16,), jnp.float32)

    def _zrow(e, carry):
        for dblk in range(D // 16):
            rows_v[e, pl.ds(dblk * 16, 16)] = zero16
        return carry

    lax.fori_loop(0, C, _zrow, 0)
    for off in range(0, ROWS_PER_SUB, C):
        sz = min(C, ROWS_PER_SUB - off)
        pltpu.sync_copy(rows_v.at[pl.ds(0, sz)],
                        accum.at[pl.ds(sid * ROWS_PER_SUB + off, sz)])

    @pl.when(sid == 0)
    def _ztail():
        pltpu.sync_copy(rows_v.at[pl.ds(0, TAIL_ROWS)],
                        accum.at[pl.ds(TAIL_OFF, TAIL_ROWS)])

    plsc.subcore_barrier()

    def _chunk(i, carry):
        c = i * NW + wid

        @pl.when(c < NCHUNK)
        def _():
            base = pl.multiple_of(c * C, C)
            pltpu.sync_copy(src_hbm.at[pl.ds(base, C)], src_v)
            pltpu.sync_copy(dst_hbm.at[pl.ds(base, C)], dst_v)
            pltpu.sync_copy(ewb_hbm.at[pl.ds(base, C)], w16_v)
            pltpu.async_copy(xn_hbm.at[src_v], rows_v, sem).wait()

            def _scale(e, carry2):
                w = w16_v[e, :]
                for dblk in range(D // 16):
                    sl = pl.ds(dblk * 16, 16)
                    rows_v[e, sl] = rows_v[e, sl] * w
                return carry2

            lax.fori_loop(0, C, _scale, 0)
            pltpu.sync_copy(rows_v, accum.at[dst_v], add=True)

        return carry

    lax.fori_loop(0, CHUNK_ITERS, _chunk, 0)
    plsc.subcore_barrier()

    pltpu.sync_copy(accum.at[pl.ds(sid * ROWS_PER_SUB, ROWS_PER_SUB)],
                    out_hbm.at[pl.ds(cid * N + sid * ROWS_PER_SUB, ROWS_PER_SUB)])

    @pl.when(sid == 0)
    def _wtail():
        pltpu.sync_copy(accum.at[pl.ds(TAIL_OFF, TAIL_ROWS)],
                        out_hbm.at[pl.ds(cid * N + TAIL_OFF, TAIL_ROWS)])


_sc_agg = pl.kernel(
    _sc_agg_body,
    out_type=jax.ShapeDtypeStruct((2 * N, D), jnp.float32),
    mesh=plsc.VectorSubcoreMesh(core_axis_name="c", subcore_axis_name="s",
                                num_cores=NC, num_subcores=NS),
    scratch_types=[
        pltpu.VMEM((C,), jnp.int32),
        pltpu.VMEM((C,), jnp.int32),
        pltpu.VMEM((C, 16), jnp.float32),
        pltpu.VMEM((C, D), jnp.float32),
        pltpu.VMEM_SHARED((N, D), jnp.float32),
        pltpu.SemaphoreType.DMA,
    ],
)


def _ln(x, g, b):
    mu = jnp.mean(x, axis=-1, keepdims=True)
    var = jnp.mean((x - mu) ** 2, axis=-1, keepdims=True)
    return (x - mu) / jnp.sqrt(var + 1e-5) * g + b


def _ln_kernel(x_ref, g_ref, b_ref, o_ref):
    o_ref[...] = _ln(x_ref[...], g_ref[...], b_ref[...])


def _mid_kernel(p0_ref, p1_ref, x_ref, w_ref, b_ref, g_ref, bln_ref,
                xnew_ref, xn_ref):
    agg = p0_ref[...] + p1_ref[...]
    h = jnp.dot(agg, w_ref[...], preferred_element_type=jnp.float32) + b_ref[...]
    xnew = x_ref[...] + jnp.maximum(h, 0.0)
    xnew_ref[...] = xnew
    xn_ref[...] = _ln(xnew, g_ref[...], bln_ref[...])


def _final_kernel(p0_ref, p1_ref, x_ref, w_ref, b_ref, wp_ref, bp_ref, o_ref):
    agg = p0_ref[...] + p1_ref[...]
    h = jnp.dot(agg, w_ref[...], preferred_element_type=jnp.float32) + b_ref[...]
    xnew = x_ref[...] + jnp.maximum(h, 0.0)
    o_ref[...] = (jnp.dot(xnew, wp_ref[...], preferred_element_type=jnp.float32)
                  + bp_ref[...])


_row_spec = pl.BlockSpec((BLK, D), lambda i: (i, 0))
_p1_spec = pl.BlockSpec((BLK, D), lambda i: (i + GRID, 0))
_vec_spec = pl.BlockSpec((1, D), lambda i: (0, 0))
_mat_spec = pl.BlockSpec((D, D), lambda i: (0, 0))


_ln_call = pl.pallas_call(
    _ln_kernel,
    grid=(GRID,),
    in_specs=[_row_spec, _vec_spec, _vec_spec],
    out_specs=_row_spec,
    out_shape=jax.ShapeDtypeStruct((N, D), jnp.float32),
)

_mid_call = pl.pallas_call(
    _mid_kernel,
    grid=(GRID,),
    in_specs=[_row_spec, _p1_spec, _row_spec, _mat_spec, _vec_spec,
              _vec_spec, _vec_spec],
    out_specs=[_row_spec, _row_spec],
    out_shape=[jax.ShapeDtypeStruct((N, D), jnp.float32),
               jax.ShapeDtypeStruct((N, D), jnp.float32)],
)

_final_call = pl.pallas_call(
    _final_kernel,
    grid=(GRID,),
    in_specs=[_row_spec, _p1_spec, _row_spec, _mat_spec, _vec_spec,
              _mat_spec, _vec_spec],
    out_specs=_row_spec,
    out_shape=jax.ShapeDtypeStruct((N, D), jnp.float32),
)


def kernel(x, edge_index, edge_weight,
           ln_g4, ln_b4, W4, b4,
           ln_g5, ln_b5, W5, b5,
           ln_g6, ln_b6, W6, b6,
           ln_g7, ln_b7, W7, b7,
           W_post, b_post):
    src = edge_index[0]
    dst = edge_index[1]
    ewb = jnp.broadcast_to(edge_weight[:, None], (E, 16))

    r = lambda v: v.reshape(1, D)

    xn = _ln_call(x, r(ln_g4), r(ln_b4))
    parts = _sc_agg(xn, src, dst, ewb)
    x, xn = _mid_call(parts, parts, x, W4, r(b4), r(ln_g5), r(ln_b5))
    parts = _sc_agg(xn, src, dst, ewb)
    x, xn = _mid_call(parts, parts, x, W5, r(b5), r(ln_g6), r(ln_b6))
    parts = _sc_agg(xn, src, dst, ewb)
    x, xn = _mid_call(parts, parts, x, W6, r(b6), r(ln_g7), r(ln_b7))
    parts = _sc_agg(xn, src, dst, ewb)
    out = _final_call(parts, parts, x, W7, r(b7), W_post, r(b_post))
    return out
